# baseline (speedup 1.0000x reference)
import jax
import jax.numpy as jnp
from jax.experimental import pallas as pl
from jax.experimental.pallas import tpu as pltpu

N = 4096
BR = 128
GRID = N // BR


def _body(data_ref, adj_ref, out_ref, acc_ref):
    i = pl.program_id(0)
    acc_ref[pl.ds(i * BR, BR)] = jnp.sum(adj_ref[...] * data_ref[...], axis=1)

    @pl.when(i == GRID - 1)
    def _final():
        x = acc_ref[...]
        m = jnp.max(x)
        e = jnp.exp(x - m)
        out_ref[...] = e / jnp.sum(e)


def kernel(data_input, adj_matrix):
    return pl.pallas_call(
        _body,
        grid=(GRID,),
        in_specs=[
            pl.BlockSpec((BR, N), lambda i: (i, 0)),
            pl.BlockSpec((BR, N), lambda i: (i, 0)),
        ],
        out_specs=pl.BlockSpec((N,), lambda i: (0,)),
        out_shape=jax.ShapeDtypeStruct((N,), jnp.float32),
        scratch_shapes=[pltpu.VMEM((N,), jnp.float32)],

    )(data_input, adj_matrix)


# speedup vs baseline: 1.1021x; 1.1021x over previous
import jax
import jax.numpy as jnp
from jax.experimental import pallas as pl
from jax.experimental.pallas import tpu as pltpu

N = 4096
BR = 256
GRID = N // BR


def _body(data_ref, adj_ref, out_ref, acc_ref):
    i = pl.program_id(0)
    acc_ref[pl.ds(i * BR, BR)] = jnp.sum(adj_ref[...] * data_ref[...], axis=1)

    @pl.when(i == GRID - 1)
    def _final():
        x = acc_ref[...]
        m = jnp.max(x)
        e = jnp.exp(x - m)
        out_ref[...] = e / jnp.sum(e)


def kernel(data_input, adj_matrix):
    return pl.pallas_call(
        _body,
        grid=(GRID,),
        in_specs=[
            pl.BlockSpec((BR, N), lambda i: (i, 0)),
            pl.BlockSpec((BR, N), lambda i: (i, 0)),
        ],
        out_specs=pl.BlockSpec((N,), lambda i: (0,)),
        out_shape=jax.ShapeDtypeStruct((N,), jnp.float32),
        scratch_shapes=[pltpu.VMEM((N,), jnp.float32)],

    )(data_input, adj_matrix)
